# baseline (speedup 1.0000x reference)
import functools
import jax
import jax.numpy as jnp
from jax import lax
from jax.experimental import pallas as pl
from jax.experimental.pallas import tpu as pltpu
from jax.experimental.pallas import tpu_sc as plsc

BATCH = 16384
SEQ = 5
DIM = 64
LANES = 16
NC, NS = 2, 16
NW = NC * NS
BPW = BATCH // NW
G = 128
KG = BPW // G
PARAMS_LEN = 336


def _sc_body(x3_hbm, table_hbm, params_hbm, out_hbm,
             idx_v, buf0, buf1, dw_v, out_v, sem0, sem1):
    wid = lax.axis_index("s") * NC + lax.axis_index("c")

    pltpu.sync_copy(params_hbm, dw_v)
    for l in range(SEQ):
        pltpu.sync_copy(x3_hbm.at[l, pl.ds(wid * KG, KG)], idx_v.at[l])

    bufs = (buf0, buf1)
    sems = (sem0, sem1)
    groups = [(k, l) for k in range(KG) for l in range(SEQ)]

    iota = lax.iota(jnp.int32, LANES)
    row_idx = [iota + ig * LANES for ig in range(G // LANES)]
    zeros = jnp.zeros((LANES,), jnp.float32)
    ones = zeros + 1.0

    copies = {}
    copies[0] = pltpu.async_copy(
        table_hbm.at[idx_v.at[groups[0][1], groups[0][0]]], bufs[0], sems[0])

    for gi, (k, l) in enumerate(groups):
        if gi + 1 < len(groups):
            k2, l2 = groups[gi + 1]
            copies[gi + 1] = pltpu.async_copy(
                table_hbm.at[idx_v.at[l2, k2]],
                bufs[(gi + 1) % 2], sems[(gi + 1) % 2])
        copies[gi].wait()
        buf = bufs[gi % 2]

        if l == 0:
            accs = [zeros] * (G // LANES)

        def dot_step(d, accs_t, l=l, buf=buf):
            accs_t = list(accs_t)
            col = jnp.zeros((LANES,), jnp.int32) + d
            wv = dw_v[l * DIM + d]
            for ig in range(G // LANES):
                v = plsc.load_gather(buf, [row_idx[ig], col])
                accs_t[ig] = accs_t[ig] + v * wv
            return tuple(accs_t)

        accs = list(lax.fori_loop(0, DIM, dot_step, tuple(accs)))

        if l == SEQ - 1:
            db = dw_v[SEQ * DIM]
            for ig in range(G // LANES):
                delta = accs[ig] + db
                p0 = ones / (ones + jnp.exp(-delta))
                off = k * G + ig * LANES
                out_v[0, pl.ds(off, LANES)] = p0
                out_v[1, pl.ds(off, LANES)] = ones - p0

    base = wid * BPW
    pltpu.sync_copy(out_v.at[0], out_hbm.at[0, pl.ds(base, BPW)])
    pltpu.sync_copy(out_v.at[1], out_hbm.at[1, pl.ds(base, BPW)])


@jax.jit
def _run(x3, table, params):
    mesh = plsc.VectorSubcoreMesh(core_axis_name="c", subcore_axis_name="s")
    f = pl.kernel(
        _sc_body,
        out_type=jax.ShapeDtypeStruct((2, BATCH), jnp.float32),
        mesh=mesh,
        scratch_types=[
            pltpu.VMEM((SEQ, KG, G), jnp.int32),
            pltpu.VMEM((G, DIM), jnp.float32),
            pltpu.VMEM((G, DIM), jnp.float32),
            pltpu.VMEM((PARAMS_LEN, LANES), jnp.float32),
            pltpu.VMEM((2, BPW), jnp.float32),
            pltpu.SemaphoreType.DMA,
            pltpu.SemaphoreType.DMA,
        ],
        compiler_params=pltpu.CompilerParams(
            needs_layout_passes=False, use_tc_tiling_on_sc=False),
    )
    return f(x3, table, params)


def kernel(x, table, W, b):
    x3 = x.astype(jnp.int32).T.reshape(SEQ, BATCH // G, G)
    params = jnp.zeros((PARAMS_LEN, LANES), jnp.float32)
    params = params.at[: SEQ * DIM].set((W[0] - W[1])[:, None])
    params = params.at[SEQ * DIM].set(b[0] - b[1])
    out2 = _run(x3, table, params)
    return out2.T


# speedup vs baseline: 4.4640x; 4.4640x over previous
import functools
import jax
import jax.numpy as jnp
from jax import lax
from jax.experimental import pallas as pl
from jax.experimental.pallas import tpu as pltpu
from jax.experimental.pallas import tpu_sc as plsc

BATCH = 16384
SEQ = 5
DIM = 64
NROWS = 1000000
LANES = 16
NC, NS = 2, 16
NW = NC * NS
BPW = BATCH // NW
G = 128
KG = BPW // G
BLK = 8192
PARAMS_LEN = 16


def _proj_body(wm_ref, tt_ref, *out_refs):
    res = jax.lax.dot_general(
        wm_ref[...], tt_ref[...], (((1,), (0,)), ((), ())),
        preferred_element_type=jnp.float32)
    for l, o in enumerate(out_refs):
        o[...] = res[l]


@jax.jit
def _proj(wm, tt):
    grid = (NROWS + BLK - 1) // BLK
    return pl.pallas_call(
        _proj_body,
        grid=(grid,),
        in_specs=[
            pl.BlockSpec((8, DIM), lambda i: (0, 0)),
            pl.BlockSpec((DIM, BLK), lambda i: (0, i)),
        ],
        out_specs=[pl.BlockSpec((BLK,), lambda i: (i,)) for _ in range(SEQ)],
        out_shape=[jax.ShapeDtypeStruct((NROWS,), jnp.float32)
                   for _ in range(SEQ)],
    )(wm, tt)


def _sc_body(x3_hbm, p0_hbm, p1_hbm, p2_hbm, p3_hbm, p4_hbm, params_hbm,
             out_hbm, idx_v, g_v, db_v, out_v, sem):
    wid = lax.axis_index("s") * NC + lax.axis_index("c")
    proj = (p0_hbm, p1_hbm, p2_hbm, p3_hbm, p4_hbm)

    pltpu.sync_copy(params_hbm, db_v)
    for l in range(SEQ):
        pltpu.sync_copy(x3_hbm.at[l, pl.ds(wid * KG, KG)], idx_v.at[l])

    copies = []
    for l in range(SEQ):
        for k in range(KG):
            copies.append(pltpu.async_copy(
                proj[l].at[idx_v.at[l, k]], g_v.at[l, k], sem))
    for c in copies:
        c.wait()

    db = db_v[...]
    ones = jnp.zeros((LANES,), jnp.float32) + 1.0
    for k in range(KG):
        for ig in range(G // LANES):
            sl = pl.ds(ig * LANES, LANES)
            delta = g_v[0, k, sl] + g_v[1, k, sl] + g_v[2, k, sl] \
                + g_v[3, k, sl] + g_v[4, k, sl] + db
            p0 = ones / (ones + jnp.exp(-delta))
            off = k * G + ig * LANES
            out_v[0, pl.ds(off, LANES)] = p0
            out_v[1, pl.ds(off, LANES)] = ones - p0

    base = wid * BPW
    pltpu.sync_copy(out_v.at[0], out_hbm.at[0, pl.ds(base, BPW)])
    pltpu.sync_copy(out_v.at[1], out_hbm.at[1, pl.ds(base, BPW)])


@jax.jit
def _run(x3, p0, p1, p2, p3, p4, params):
    mesh = plsc.VectorSubcoreMesh(core_axis_name="c", subcore_axis_name="s")
    f = pl.kernel(
        _sc_body,
        out_type=jax.ShapeDtypeStruct((2, BATCH), jnp.float32),
        mesh=mesh,
        scratch_types=[
            pltpu.VMEM((SEQ, KG, G), jnp.int32),
            pltpu.VMEM((SEQ, KG, G), jnp.float32),
            pltpu.VMEM((PARAMS_LEN,), jnp.float32),
            pltpu.VMEM((2, BPW), jnp.float32),
            pltpu.SemaphoreType.DMA,
        ],
        compiler_params=pltpu.CompilerParams(
            needs_layout_passes=False, use_tc_tiling_on_sc=False),
    )
    return f(x3, p0, p1, p2, p3, p4, params)


def kernel(x, table, W, b):
    dw = W[0] - W[1]
    wm = jnp.zeros((8, DIM), jnp.float32).at[:SEQ].set(dw.reshape(SEQ, DIM))
    projs = _proj(wm, table.T)
    x3 = x.astype(jnp.int32).T.reshape(SEQ, BATCH // G, G)
    params = jnp.full((PARAMS_LEN,), b[0] - b[1], jnp.float32)
    out2 = _run(x3, *projs, params)
    return out2.T


# speedup vs baseline: 5.7633x; 1.2910x over previous
import functools
import jax
import jax.numpy as jnp
from jax import lax
from jax.experimental import pallas as pl
from jax.experimental.pallas import tpu as pltpu
from jax.experimental.pallas import tpu_sc as plsc

BATCH = 16384
SEQ = 5
DIM = 64
NROWS = 1000000
LANES = 16
NC, NS = 2, 16
NW = NC * NS
BPW = BATCH // NW
G = 128
KG = BPW // G
BLK = 16384
PARAMS_LEN = 16


def _proj_body(wm_ref, tt_ref, *out_refs):
    res = jax.lax.dot_general(
        wm_ref[...], tt_ref[...], (((1,), (0,)), ((), ())),
        preferred_element_type=jnp.float32)
    for l, o in enumerate(out_refs):
        o[...] = res[l]


@jax.jit
def _proj(wm, tt):
    grid = (NROWS + BLK - 1) // BLK
    return pl.pallas_call(
        _proj_body,
        grid=(grid,),
        in_specs=[
            pl.BlockSpec((8, DIM), lambda i: (0, 0)),
            pl.BlockSpec((DIM, BLK), lambda i: (0, i)),
        ],
        out_specs=[pl.BlockSpec((BLK,), lambda i: (i,)) for _ in range(SEQ)],
        out_shape=[jax.ShapeDtypeStruct((NROWS,), jnp.float32)
                   for _ in range(SEQ)],
    )(wm, tt)


def _sc_body(x3_hbm, p0_hbm, p1_hbm, p2_hbm, p3_hbm, p4_hbm, params_hbm,
             out_hbm, idx_v, g_v, db_v, out_v, sem):
    wid = lax.axis_index("s") * NC + lax.axis_index("c")
    proj = (p0_hbm, p1_hbm, p2_hbm, p3_hbm, p4_hbm)

    pltpu.sync_copy(params_hbm, db_v)
    for l in range(SEQ):
        pltpu.sync_copy(x3_hbm.at[l, pl.ds(wid * KG, KG)], idx_v.at[l])

    copies = []
    for l in range(SEQ):
        for k in range(KG):
            copies.append(pltpu.async_copy(
                proj[l].at[idx_v.at[l, k]], g_v.at[l, k], sem))
    for c in copies:
        c.wait()

    db = db_v[...]
    ones = jnp.zeros((LANES,), jnp.float32) + 1.0
    for k in range(KG):
        for ig in range(G // LANES):
            sl = pl.ds(ig * LANES, LANES)
            delta = g_v[0, k, sl] + g_v[1, k, sl] + g_v[2, k, sl] \
                + g_v[3, k, sl] + g_v[4, k, sl] + db
            p0 = ones / (ones + jnp.exp(-delta))
            off = k * G + ig * LANES
            out_v[0, pl.ds(off, LANES)] = p0
            out_v[1, pl.ds(off, LANES)] = ones - p0

    base = wid * BPW
    pltpu.sync_copy(out_v.at[0], out_hbm.at[0, pl.ds(base, BPW)])
    pltpu.sync_copy(out_v.at[1], out_hbm.at[1, pl.ds(base, BPW)])


@jax.jit
def _run(x3, p0, p1, p2, p3, p4, params):
    mesh = plsc.VectorSubcoreMesh(core_axis_name="c", subcore_axis_name="s")
    f = pl.kernel(
        _sc_body,
        out_type=jax.ShapeDtypeStruct((2, BATCH), jnp.float32),
        mesh=mesh,
        scratch_types=[
            pltpu.VMEM((SEQ, KG, G), jnp.int32),
            pltpu.VMEM((SEQ, KG, G), jnp.float32),
            pltpu.VMEM((PARAMS_LEN,), jnp.float32),
            pltpu.VMEM((2, BPW), jnp.float32),
            pltpu.SemaphoreType.DMA,
        ],
        compiler_params=pltpu.CompilerParams(
            needs_layout_passes=False, use_tc_tiling_on_sc=False),
    )
    return f(x3, p0, p1, p2, p3, p4, params)


def kernel(x, table, W, b):
    dw = W[0] - W[1]
    wm = jnp.zeros((8, DIM), jnp.float32).at[:SEQ].set(dw.reshape(SEQ, DIM))
    projs = _proj(wm, table.T)
    x3 = x.astype(jnp.int32).T.reshape(SEQ, BATCH // G, G)
    params = jnp.full((PARAMS_LEN,), b[0] - b[1], jnp.float32)
    out2 = _run(x3, *projs, params)
    return out2.T


# speedup vs baseline: 6.2452x; 1.0836x over previous
import functools
import jax
import jax.numpy as jnp
from jax import lax
from jax.experimental import pallas as pl
from jax.experimental.pallas import tpu as pltpu
from jax.experimental.pallas import tpu_sc as plsc

BATCH = 16384
SEQ = 5
DIM = 64
NROWS = 1000000
LANES = 16
NC, NS = 2, 16
NW = NC * NS
BPW = BATCH // NW
G = 128
KG = BPW // G
BLK = 32768
PARAMS_LEN = 16


def _proj_body(wm_ref, tt_ref, *out_refs):
    res = jax.lax.dot_general(
        wm_ref[...], tt_ref[...], (((1,), (0,)), ((), ())),
        preferred_element_type=jnp.float32)
    for l, o in enumerate(out_refs):
        o[...] = res[l]


@jax.jit
def _proj(wm, tt):
    grid = (NROWS + BLK - 1) // BLK
    return pl.pallas_call(
        _proj_body,
        grid=(grid,),
        in_specs=[
            pl.BlockSpec((8, DIM), lambda i: (0, 0)),
            pl.BlockSpec((DIM, BLK), lambda i: (0, i)),
        ],
        out_specs=[pl.BlockSpec((BLK,), lambda i: (i,)) for _ in range(SEQ)],
        out_shape=[jax.ShapeDtypeStruct((NROWS,), jnp.float32)
                   for _ in range(SEQ)],
    )(wm, tt)


def _sc_body(x3_hbm, p0_hbm, p1_hbm, p2_hbm, p3_hbm, p4_hbm, params_hbm,
             out_hbm, idx_v, g_v, db_v, out_v, sem):
    wid = lax.axis_index("s") * NC + lax.axis_index("c")
    proj = (p0_hbm, p1_hbm, p2_hbm, p3_hbm, p4_hbm)

    pltpu.sync_copy(params_hbm, db_v)
    for l in range(SEQ):
        pltpu.sync_copy(x3_hbm.at[l, pl.ds(wid * KG, KG)], idx_v.at[l])

    copies = []
    for l in range(SEQ):
        for k in range(KG):
            copies.append(pltpu.async_copy(
                proj[l].at[idx_v.at[l, k]], g_v.at[l, k], sem))
    for c in copies:
        c.wait()

    db = db_v[...]
    ones = jnp.zeros((LANES,), jnp.float32) + 1.0
    for k in range(KG):
        for ig in range(G // LANES):
            sl = pl.ds(ig * LANES, LANES)
            delta = g_v[0, k, sl] + g_v[1, k, sl] + g_v[2, k, sl] \
                + g_v[3, k, sl] + g_v[4, k, sl] + db
            p0 = ones / (ones + jnp.exp(-delta))
            off = k * G + ig * LANES
            out_v[0, pl.ds(off, LANES)] = p0
            out_v[1, pl.ds(off, LANES)] = ones - p0

    base = wid * BPW
    pltpu.sync_copy(out_v.at[0], out_hbm.at[0, pl.ds(base, BPW)])
    pltpu.sync_copy(out_v.at[1], out_hbm.at[1, pl.ds(base, BPW)])


@jax.jit
def _run(x3, p0, p1, p2, p3, p4, params):
    mesh = plsc.VectorSubcoreMesh(core_axis_name="c", subcore_axis_name="s")
    f = pl.kernel(
        _sc_body,
        out_type=jax.ShapeDtypeStruct((2, BATCH), jnp.float32),
        mesh=mesh,
        scratch_types=[
            pltpu.VMEM((SEQ, KG, G), jnp.int32),
            pltpu.VMEM((SEQ, KG, G), jnp.float32),
            pltpu.VMEM((PARAMS_LEN,), jnp.float32),
            pltpu.VMEM((2, BPW), jnp.float32),
            pltpu.SemaphoreType.DMA,
        ],
        compiler_params=pltpu.CompilerParams(
            needs_layout_passes=False, use_tc_tiling_on_sc=False),
    )
    return f(x3, p0, p1, p2, p3, p4, params)


def kernel(x, table, W, b):
    dw = W[0] - W[1]
    wm = jnp.zeros((8, DIM), jnp.float32).at[:SEQ].set(dw.reshape(SEQ, DIM))
    projs = _proj(wm, table.T)
    x3 = x.astype(jnp.int32).T.reshape(SEQ, BATCH // G, G)
    params = jnp.full((PARAMS_LEN,), b[0] - b[1], jnp.float32)
    out2 = _run(x3, *projs, params)
    return out2.T
